# Initial kernel scaffold; baseline (speedup 1.0000x reference)
import jax
import jax.numpy as jnp
from jax.experimental import pallas as pl


def kernel(x, edge_index, Ws, Wn_in, Wn_out, Wg):
    raise NotImplementedError("write your pallas kernel here")


# speedup vs baseline: 3.4929x; 3.4929x over previous
import functools

import jax
import jax.numpy as jnp
from jax import lax
from jax.experimental import pallas as pl
from jax.experimental.pallas import tpu as pltpu
from jax.experimental.pallas import tpu_sc as plsc

N = 10000
E = 320000
D = 128
NC = 2
NS = 16
NW = NC * NS
EPW = E // NW
CHUNK = 80
NCHUNK = EPW // CHUNK
NPAD = 10240
RPT = NPAD // NS


def _sc_agg_body(with_deg, h_hbm, src_hbm, dst_hbm, *refs):
    if with_deg:
        (ain_hbm, aout_hbm, din_hbm, dout_hbm,
         acc_sh, sidx, didx, rows, sem) = refs
    else:
        (ain_hbm, aout_hbm,
         acc_sh, sidx, didx, rows, sem) = refs

    cid = lax.axis_index("c")
    sid = lax.axis_index("s")
    wid = cid * NS + sid
    ebase = wid * EPW
    r0 = sid * RPT

    def _fill_rows(val):
        vv = jnp.full((16,), val, jnp.float32)

        def _fill(i, c):
            for jj in range(D // 16):
                rows[i, pl.ds(jj * 16, 16)] = vv
            return c
        lax.fori_loop(0, CHUNK, _fill, 0)

    def _zero_acc():
        _fill_rows(0.0)
        for k in range(RPT // CHUNK):
            pltpu.sync_copy(rows, acc_sh.at[pl.ds(r0 + k * CHUNK, CHUNK)])

    def _accumulate(gidx_hbm, scatter_hbm):
        def _body(j, c):
            base = ebase + j * CHUNK
            pltpu.sync_copy(gidx_hbm.at[pl.ds(base, CHUNK)], sidx)
            pltpu.sync_copy(scatter_hbm.at[pl.ds(base, CHUNK)], didx)
            pltpu.async_copy(h_hbm.at[sidx], rows, sem).wait()
            pltpu.sync_copy(rows, acc_sh.at[didx], add=True)
            return c
        lax.fori_loop(0, NCHUNK, _body, 0)

    def _count(scatter_hbm):
        def _body(j, c):
            base = ebase + j * CHUNK
            pltpu.sync_copy(scatter_hbm.at[pl.ds(base, CHUNK)], didx)
            pltpu.sync_copy(rows, acc_sh.at[didx], add=True)
            return c
        lax.fori_loop(0, NCHUNK, _body, 0)

    def _writeout(out_hbm):
        pltpu.sync_copy(acc_sh.at[pl.ds(r0, RPT)],
                        out_hbm.at[cid, pl.ds(r0, RPT)])

    _zero_acc()
    plsc.subcore_barrier()
    _accumulate(src_hbm, dst_hbm)
    plsc.subcore_barrier()
    _writeout(ain_hbm)
    _zero_acc()
    plsc.subcore_barrier()

    _accumulate(dst_hbm, src_hbm)
    plsc.subcore_barrier()
    _writeout(aout_hbm)

    if with_deg:
        _zero_acc()
        _fill_rows(1.0)
        plsc.subcore_barrier()
        _count(dst_hbm)
        plsc.subcore_barrier()
        _writeout(din_hbm)
        _zero_acc()
        _fill_rows(1.0)
        plsc.subcore_barrier()
        _count(src_hbm)
        plsc.subcore_barrier()
        _writeout(dout_hbm)


def _make_sc_agg(with_deg):
    n_out = 4 if with_deg else 2
    out_type = tuple(jax.ShapeDtypeStruct((NC, NPAD, D), jnp.float32)
                     for _ in range(n_out))
    scratch = [pltpu.VMEM_SHARED((NPAD, D), jnp.float32),
               pltpu.VMEM((CHUNK,), jnp.int32),
               pltpu.VMEM((CHUNK,), jnp.int32),
               pltpu.VMEM((CHUNK, D), jnp.float32),
               pltpu.SemaphoreType.DMA]
    mesh = plsc.VectorSubcoreMesh(core_axis_name="c", subcore_axis_name="s")
    return pl.kernel(functools.partial(_sc_agg_body, with_deg),
                     out_type=out_type, mesh=mesh,
                     scratch_types=scratch)


_sc_agg_deg = _make_sc_agg(True)
_sc_agg = _make_sc_agg(False)

BN = 400
GRID = N // BN


def _fuse_body(h_ref, ain_ref, aout_ref, din_ref, dout_ref,
               ws_ref, wni_ref, wno_ref, wg_ref, out_ref):
    f32 = jnp.float32
    h = h_ref[...]
    din = jnp.maximum(din_ref[0, :, 0:1] + din_ref[1, :, 0:1], 1.0)
    dout = jnp.maximum(dout_ref[0, :, 0:1] + dout_ref[1, :, 0:1], 1.0)
    a_in = (ain_ref[0] + ain_ref[1]) / din
    a_out = (aout_ref[0] + aout_ref[1]) / dout
    hs = jnp.dot(h, ws_ref[...], preferred_element_type=f32)
    h_in = hs + jnp.dot(a_in, wni_ref[...], preferred_element_type=f32)
    h_out = hs + jnp.dot(a_out, wno_ref[...], preferred_element_type=f32)
    g = (jnp.dot(h_in, wg_ref[0:D, :], preferred_element_type=f32)
         + jnp.dot(h_out, wg_ref[D:2 * D, :], preferred_element_type=f32))
    z = jax.nn.sigmoid(g)
    out_ref[...] = jnp.maximum(z * h_in + (1.0 - z) * h_out, 0.0)


_fuse = pl.pallas_call(
    _fuse_body,
    grid=(GRID,),
    in_specs=[
        pl.BlockSpec((BN, D), lambda i: (i, 0)),
        pl.BlockSpec((NC, BN, D), lambda i: (0, i, 0)),
        pl.BlockSpec((NC, BN, D), lambda i: (0, i, 0)),
        pl.BlockSpec((NC, BN, D), lambda i: (0, i, 0)),
        pl.BlockSpec((NC, BN, D), lambda i: (0, i, 0)),
        pl.BlockSpec((D, D), lambda i: (0, 0)),
        pl.BlockSpec((D, D), lambda i: (0, 0)),
        pl.BlockSpec((D, D), lambda i: (0, 0)),
        pl.BlockSpec((2 * D, D), lambda i: (0, 0)),
    ],
    out_specs=pl.BlockSpec((BN, D), lambda i: (i, 0)),
    out_shape=jax.ShapeDtypeStruct((N, D), jnp.float32),
)


def kernel(x, edge_index, Ws, Wn_in, Wn_out, Wg):
    src = edge_index[0]
    dst = edge_index[1]
    ain, aout, din, dout = _sc_agg_deg(x, src, dst)
    h1 = _fuse(x, ain, aout, din, dout, Ws[0], Wn_in[0], Wn_out[0], Wg[0])
    ain2, aout2 = _sc_agg(h1, src, dst)
    h2 = _fuse(h1, ain2, aout2, din, dout, Ws[1], Wn_in[1], Wn_out[1], Wg[1])
    return h2


# speedup vs baseline: 5.1837x; 1.4841x over previous
import functools

import jax
import jax.numpy as jnp
from jax import lax
from jax.experimental import pallas as pl
from jax.experimental.pallas import tpu as pltpu
from jax.experimental.pallas import tpu_sc as plsc

N = 10000
E = 320000
D = 128
NC = 2
NS = 16
NW = NC * NS
EPW = E // NW
CHUNK = 80
NCHUNK = EPW // CHUNK
NPAD = 10240
RPT = NPAD // NS


def _sc_agg_body(with_deg, h_hbm, src_hbm, dst_hbm, *refs):
    if with_deg:
        (ain_hbm, aout_hbm, din_hbm, dout_hbm,
         acc_sh, sidx0, sidx1, didx0, didx1, rows, rows1, sem0, sem1) = refs
    else:
        (ain_hbm, aout_hbm,
         acc_sh, sidx0, sidx1, didx0, didx1, rows, rows1, sem0, sem1) = refs
    sidx = sidx0
    didx = didx0

    cid = lax.axis_index("c")
    sid = lax.axis_index("s")
    wid = cid * NS + sid
    ebase = wid * EPW
    r0 = sid * RPT

    def _fill_rows(val):
        vv = jnp.full((16,), val, jnp.float32)

        def _fill(i, c):
            for jj in range(D // 16):
                rows[i, pl.ds(jj * 16, 16)] = vv
            return c
        lax.fori_loop(0, CHUNK, _fill, 0)

    def _zero_acc():
        _fill_rows(0.0)
        for k in range(RPT // CHUNK):
            pltpu.sync_copy(rows, acc_sh.at[pl.ds(r0 + k * CHUNK, CHUNK)])

    def _accumulate(gidx_hbm, scatter_hbm):
        bufs = ((sidx0, didx0, rows, sem0), (sidx1, didx1, rows1, sem1))

        def _start(k, b):
            si, di, rw, sm = bufs[b]
            base = ebase + k * CHUNK
            pltpu.sync_copy(gidx_hbm.at[pl.ds(base, CHUNK)], si)
            pltpu.sync_copy(scatter_hbm.at[pl.ds(base, CHUNK)], di)
            pltpu.async_copy(h_hbm.at[si], rw, sm)

        def _drain(b):
            si, di, rw, sm = bufs[b]
            pltpu.make_async_copy(h_hbm.at[si], rw, sm).wait()
            pltpu.sync_copy(rw, acc_sh.at[di], add=True)

        _start(0, 0)

        def _body(i, c):
            _start(2 * i + 1, 1)
            _drain(0)
            _start(2 * i + 2, 0)
            _drain(1)
            return c
        lax.fori_loop(0, (NCHUNK - 1) // 2, _body, 0)
        _drain(0)

    def _count(scatter_hbm):
        def _body(j, c):
            base = ebase + j * CHUNK
            pltpu.sync_copy(scatter_hbm.at[pl.ds(base, CHUNK)], didx)
            pltpu.sync_copy(rows, acc_sh.at[didx], add=True)
            return c
        lax.fori_loop(0, NCHUNK, _body, 0)

    def _writeout(out_hbm):
        pltpu.sync_copy(acc_sh.at[pl.ds(r0, RPT)],
                        out_hbm.at[cid, pl.ds(r0, RPT)])

    _zero_acc()
    plsc.subcore_barrier()
    _accumulate(src_hbm, dst_hbm)
    plsc.subcore_barrier()
    _writeout(ain_hbm)
    _zero_acc()
    plsc.subcore_barrier()

    _accumulate(dst_hbm, src_hbm)
    plsc.subcore_barrier()
    _writeout(aout_hbm)

    if with_deg:
        _zero_acc()
        _fill_rows(1.0)
        plsc.subcore_barrier()
        _count(dst_hbm)
        plsc.subcore_barrier()
        _writeout(din_hbm)
        _zero_acc()
        _fill_rows(1.0)
        plsc.subcore_barrier()
        _count(src_hbm)
        plsc.subcore_barrier()
        _writeout(dout_hbm)


def _make_sc_agg(with_deg):
    n_out = 4 if with_deg else 2
    out_type = tuple(jax.ShapeDtypeStruct((NC, NPAD, D), jnp.float32)
                     for _ in range(n_out))
    scratch = [pltpu.VMEM_SHARED((NPAD, D), jnp.float32),
               pltpu.VMEM((CHUNK,), jnp.int32),
               pltpu.VMEM((CHUNK,), jnp.int32),
               pltpu.VMEM((CHUNK,), jnp.int32),
               pltpu.VMEM((CHUNK,), jnp.int32),
               pltpu.VMEM((CHUNK, D), jnp.float32),
               pltpu.VMEM((CHUNK, D), jnp.float32),
               pltpu.SemaphoreType.DMA,
               pltpu.SemaphoreType.DMA]
    mesh = plsc.VectorSubcoreMesh(core_axis_name="c", subcore_axis_name="s")
    return pl.kernel(functools.partial(_sc_agg_body, with_deg),
                     out_type=out_type, mesh=mesh,
                     scratch_types=scratch)


_sc_agg_deg = _make_sc_agg(True)
_sc_agg = _make_sc_agg(False)

BN = 400
GRID = N // BN


def _fuse_body(h_ref, ain_ref, aout_ref, din_ref, dout_ref,
               ws_ref, wni_ref, wno_ref, wg_ref, out_ref):
    f32 = jnp.float32
    h = h_ref[...]
    din = jnp.maximum(din_ref[0, :, 0:1] + din_ref[1, :, 0:1], 1.0)
    dout = jnp.maximum(dout_ref[0, :, 0:1] + dout_ref[1, :, 0:1], 1.0)
    a_in = (ain_ref[0] + ain_ref[1]) / din
    a_out = (aout_ref[0] + aout_ref[1]) / dout
    hs = jnp.dot(h, ws_ref[...], preferred_element_type=f32)
    h_in = hs + jnp.dot(a_in, wni_ref[...], preferred_element_type=f32)
    h_out = hs + jnp.dot(a_out, wno_ref[...], preferred_element_type=f32)
    g = (jnp.dot(h_in, wg_ref[0:D, :], preferred_element_type=f32)
         + jnp.dot(h_out, wg_ref[D:2 * D, :], preferred_element_type=f32))
    z = jax.nn.sigmoid(g)
    out_ref[...] = jnp.maximum(z * h_in + (1.0 - z) * h_out, 0.0)


_fuse = pl.pallas_call(
    _fuse_body,
    grid=(GRID,),
    in_specs=[
        pl.BlockSpec((BN, D), lambda i: (i, 0)),
        pl.BlockSpec((NC, BN, D), lambda i: (0, i, 0)),
        pl.BlockSpec((NC, BN, D), lambda i: (0, i, 0)),
        pl.BlockSpec((NC, BN, D), lambda i: (0, i, 0)),
        pl.BlockSpec((NC, BN, D), lambda i: (0, i, 0)),
        pl.BlockSpec((D, D), lambda i: (0, 0)),
        pl.BlockSpec((D, D), lambda i: (0, 0)),
        pl.BlockSpec((D, D), lambda i: (0, 0)),
        pl.BlockSpec((2 * D, D), lambda i: (0, 0)),
    ],
    out_specs=pl.BlockSpec((BN, D), lambda i: (i, 0)),
    out_shape=jax.ShapeDtypeStruct((N, D), jnp.float32),
)


def kernel(x, edge_index, Ws, Wn_in, Wn_out, Wg):
    src = edge_index[0]
    dst = edge_index[1]
    ain, aout, din, dout = _sc_agg_deg(x, src, dst)
    h1 = _fuse(x, ain, aout, din, dout, Ws[0], Wn_in[0], Wn_out[0], Wg[0])
    ain2, aout2 = _sc_agg(h1, src, dst)
    h2 = _fuse(h1, ain2, aout2, din, dout, Ws[1], Wn_in[1], Wn_out[1], Wg[1])
    return h2
